# Initial kernel scaffold; baseline (speedup 1.0000x reference)
import jax
import jax.numpy as jnp
from jax.experimental import pallas as pl


def kernel(x, edge_index, W1, b1, W2, b2, W3, b3, W4, b4):
    raise NotImplementedError("write your pallas kernel here")


# speedup vs baseline: 7.6860x; 7.6860x over previous
import functools

import jax
import jax.numpy as jnp
from jax import lax
from jax.experimental import pallas as pl
from jax.experimental.pallas import tpu as pltpu
from jax.experimental.pallas import tpu_sc as plsc

NC = 2
NS = 16
NW = NC * NS
CH = 128
N = 10000
NPAD = 10240
HR = NPAD // 128
E = 320000

_mesh = plsc.VectorSubcoreMesh(core_axis_name="c", subcore_axis_name="s")
_f32 = jnp.float32


def _zero16():
    return jnp.zeros((16,), _f32)


def _ones16():
    return jnp.ones((16,), _f32)


def _deg_body(ew, dst_hbm, out_hbm, dstv, hist):
    cid = lax.axis_index("c")
    sid = lax.axis_index("s")
    wid = sid * NC + cid

    def _zrow(i, c):
        hist[pl.ds(pl.multiple_of(i * 16, 8), 16)] = _zero16()
        return c

    lax.fori_loop(0, NPAD // 16, _zrow, 0)

    def _chunk(c, carry):
        off = pl.multiple_of(wid * ew + c * CH, 8)
        pltpu.sync_copy(dst_hbm.at[pl.ds(off, CH)], dstv)
        for j in range(CH // 16):
            dv = dstv[pl.ds(j * 16, 16)]
            plsc.addupdate_scatter(hist, [dv], _ones16())
        return carry

    lax.fori_loop(0, ew // CH, _chunk, 0)

    pltpu.sync_copy(hist,
                    out_hbm.at[pl.ds(pl.multiple_of(wid * NPAD, 8), NPAD)])


def _agg_body(ew, g_hbm, src_hbm, dst_hbm, out_hbm, srcv, dstv, rows,
              out_sh, sem):
    cid = lax.axis_index("c")
    sid = lax.axis_index("s")
    wid = sid * NC + cid
    rows_per_tile = NPAD // NS

    def _zrow(i, c):
        for j in range(8):
            rows[i, pl.ds(j * 16, 16)] = _zero16()
        return c

    lax.fori_loop(0, CH, _zrow, 0)
    for r in range(rows_per_tile // CH):
        pltpu.sync_copy(rows, out_sh.at[pl.ds(
            pl.multiple_of(sid * rows_per_tile + r * CH, 8), CH)])
    plsc.subcore_barrier()

    def _chunk(c, carry):
        off = pl.multiple_of(wid * ew + c * CH, 8)
        pltpu.sync_copy(src_hbm.at[pl.ds(off, CH)], srcv)
        pltpu.async_copy(g_hbm.at[srcv], rows, sem).wait()
        pltpu.sync_copy(dst_hbm.at[pl.ds(off, CH)], dstv)
        pltpu.sync_copy(rows, out_sh.at[dstv], add=True)
        return carry

    lax.fori_loop(0, ew // CH, _chunk, 0)
    plsc.subcore_barrier()

    for r in range(rows_per_tile // CH):
        sl = pl.ds(pl.multiple_of(sid * rows_per_tile + r * CH, 8), CH)
        pltpu.sync_copy(out_sh.at[sl], rows)
        pltpu.sync_copy(rows, out_hbm.at[pl.ds(
            pl.multiple_of(cid * NPAD + sid * rows_per_tile + r * CH, 8),
            CH)])


def _make_deg(ew):
    return pl.kernel(
        functools.partial(_deg_body, ew),
        out_type=jax.ShapeDtypeStruct((NW * NPAD,), _f32),
        mesh=_mesh,
        scratch_types=[
            pltpu.VMEM((CH,), jnp.int32),
            pltpu.VMEM((NPAD,), _f32),
        ],
        compiler_params=pltpu.CompilerParams(needs_layout_passes=False),
    )


def _make_agg(ew):
    return pl.kernel(
        functools.partial(_agg_body, ew),
        out_type=jax.ShapeDtypeStruct((NC * NPAD, 128), _f32),
        mesh=_mesh,
        scratch_types=[
            pltpu.VMEM((CH,), jnp.int32),
            pltpu.VMEM((CH,), jnp.int32),
            pltpu.VMEM((CH, 128), _f32),
            pltpu.VMEM_SHARED((NPAD, 128), _f32),
            pltpu.SemaphoreType.DMA,
        ],
        compiler_params=pltpu.CompilerParams(needs_layout_passes=False),
    )


def _tc1_body(x_ref, w_ref, d_ref, g_ref, dis_ref):
    deg = jnp.sum(d_ref[...], axis=0) + 1.0
    disc = lax.rsqrt(deg)
    dis_ref[...] = disc
    h = jnp.dot(x_ref[...], w_ref[...], preferred_element_type=_f32)
    g_ref[...] = h * disc


def _tc_mid_body(e0_ref, e1_ref, g_ref, dis_ref, b_ref, w_ref, go_ref):
    disc = dis_ref[...]
    t = (e0_ref[...] + e1_ref[...] + g_ref[...]) * disc + b_ref[...]
    xn = jnp.maximum(t, 0.0)
    h = jnp.dot(xn, w_ref[...], preferred_element_type=_f32)
    go_ref[...] = h * disc


def _tc_fin_body(e0_ref, e1_ref, g_ref, dis_ref, b_ref, w_ref, b4_ref,
                 o_ref):
    disc = dis_ref[...]
    t = (e0_ref[...] + e1_ref[...] + g_ref[...]) * disc + b_ref[...]
    xn = jnp.maximum(t, 0.0)
    o_ref[...] = jnp.dot(xn, w_ref[...],
                         preferred_element_type=_f32) + b4_ref[...]


_GRID = NPAD // 128

_blk_rows = pl.BlockSpec((128, 128), lambda b: (b, 0))
_blk_w = pl.BlockSpec((128, 128), lambda b: (0, 0))
_blk_dis = pl.BlockSpec((128, 1), lambda b: (b, 0))
_blk_bias = pl.BlockSpec((1, 128), lambda b: (0, 0))


def _tc1(x_pad, W1, deg_all):
    return pl.pallas_call(
        _tc1_body,
        grid=(_GRID,),
        in_specs=[
            _blk_rows,
            _blk_w,
            pl.BlockSpec((NW, 128, 1), lambda b: (0, b, 0)),
        ],
        out_specs=[_blk_rows, _blk_dis],
        out_shape=[
            jax.ShapeDtypeStruct((NPAD, 128), _f32),
            jax.ShapeDtypeStruct((NPAD, 1), _f32),
        ],
    )(x_pad, W1, deg_all)


def _tc_mid(e_flat, g_prev, disp, b_row, W):
    return pl.pallas_call(
        _tc_mid_body,
        grid=(_GRID,),
        in_specs=[
            pl.BlockSpec((128, 128), lambda b: (b, 0)),
            pl.BlockSpec((128, 128), lambda b: (b + _GRID, 0)),
            _blk_rows,
            _blk_dis,
            _blk_bias,
            _blk_w,
        ],
        out_specs=_blk_rows,
        out_shape=jax.ShapeDtypeStruct((NPAD, 128), _f32),
    )(e_flat, e_flat, g_prev, disp, b_row, W)


def _tc_fin(e_flat, g_prev, disp, b_row, W4, b4_row):
    dout = W4.shape[1]
    return pl.pallas_call(
        _tc_fin_body,
        grid=(_GRID,),
        in_specs=[
            pl.BlockSpec((128, 128), lambda b: (b, 0)),
            pl.BlockSpec((128, 128), lambda b: (b + _GRID, 0)),
            _blk_rows,
            _blk_dis,
            _blk_bias,
            pl.BlockSpec((128, dout), lambda b: (0, 0)),
            pl.BlockSpec((1, dout), lambda b: (0, 0)),
        ],
        out_specs=pl.BlockSpec((128, dout), lambda b: (b, 0)),
        out_shape=jax.ShapeDtypeStruct((NPAD, dout), _f32),
    )(e_flat, e_flat, g_prev, disp, b_row, W4, b4_row)


def kernel(x, edge_index, W1, b1, W2, b2, W3, b3, W4, b4):
    src = edge_index[0]
    dst = edge_index[1]

    e_pad = ((E + NW * CH - 1) // (NW * CH)) * (NW * CH)
    padn = e_pad - E
    ew = e_pad // NW
    pad_src = jnp.zeros((padn,), jnp.int32)
    pad_dst = (N + (jnp.arange(padn, dtype=jnp.int32) % (NPAD - N)))
    src_pad = jnp.concatenate([src, pad_src])
    dst_pad = jnp.concatenate([dst, pad_dst])

    x_pad = jnp.concatenate(
        [x, jnp.zeros((NPAD - N, x.shape[1]), _f32)])

    deg_all = _make_deg(ew)(dst_pad).reshape(NW, NPAD, 1)
    g1, disp = _tc1(x_pad, W1, deg_all)

    b1r = b1.reshape(1, -1)
    b2r = b2.reshape(1, -1)
    b3r = b3.reshape(1, -1)
    b4r = b4.reshape(1, -1)

    agg = _make_agg(ew)
    e1f = agg(g1, src_pad, dst_pad)
    g2 = _tc_mid(e1f, g1, disp, b1r, W2)
    e2f = agg(g2, src_pad, dst_pad)
    g3 = _tc_mid(e2f, g2, disp, b2r, W3)
    e3f = agg(g3, src_pad, dst_pad)
    out = _tc_fin(e3f, g3, disp, b3r, W4, b4r)
    return out[:N]
